# baseline (speedup 1.0000x reference)
import functools

import jax
import jax.numpy as jnp
from jax import lax
from jax.experimental import pallas as pl
from jax.experimental.pallas import tpu as pltpu
from jax.experimental.pallas import tpu_sc as plsc

_N = 20000
_ROWS = 160
_NP = _ROWS * 128
_K = 2000
_DET = 100
_NEG = -1e9
_PAD = -3e9
_DEAD = -2e9

_NW = 32
_PER_W = _NP // _NW
_CAP = 128
_POOL = _NW * _CAP
_PROWS = _POOL // 128


def _search_body(s_ref, out_ref):
    s = s_ref[...]
    key = jnp.where(s > jnp.float32(0.05), s, jnp.float32(_NEG))
    kbv = lax.bitcast_convert_type(key, jnp.int32)
    ridx = lax.broadcasted_iota(jnp.int32, (_ROWS, 128), 0)
    lidx = lax.broadcasted_iota(jnp.int32, (_ROWS, 128), 1)
    idx = ridx * 128 + lidx

    lo0 = lax.bitcast_convert_type(jnp.float32(0.04), jnp.int32)
    hi0 = lax.bitcast_convert_type(jnp.float32(1.5), jnp.int32)

    def bs_body(_, lh):
        lo, hi = lh
        mid = (lo + hi) // 2
        cnt = jnp.sum(jnp.where(kbv >= mid, 1, 0))
        pred = cnt >= _K
        return (jnp.where(pred, mid, lo), jnp.where(pred, hi, mid))

    lo, hi = lax.fori_loop(0, 26, bs_body, (lo0, hi0))

    n_gt = jnp.sum(jnp.where(kbv > lo, 1, 0))
    ties_needed = _K - n_gt

    def ib_body(_, lh):
        lom, him = lh
        mid = (lom + him) // 2
        cnt = jnp.sum(jnp.where((kbv == lo) & (idx < mid), 1, 0))
        pred = cnt >= ties_needed
        return (jnp.where(pred, lom, mid), jnp.where(pred, mid, him))

    _, him = lax.fori_loop(0, 15, ib_body, (jnp.int32(0), jnp.int32(_NP)))

    rowi = lax.broadcasted_iota(jnp.int32, (8, 128), 0)
    lo_f = lax.bitcast_convert_type(lo, jnp.float32)
    out_ref[...] = jnp.where(rowi == 0, lo_f, him.astype(jnp.float32))


def _compact_body(s_hbm, x1_hbm, y1_hbm, x2_hbm, y2_hbm, lo_hbm, him_hbm,
                  pk_hbm, px1_hbm, py1_hbm, px2_hbm, py2_hbm,
                  s_v, x1_v, y1_v, x2_v, y2_v, th_v,
                  kb_v, bx1_v, by1_v, bx2_v, by2_v):
    wid = lax.axis_index("s") * 2 + lax.axis_index("c")
    base = wid * _PER_W
    pltpu.sync_copy(s_hbm.at[pl.ds(base, _PER_W)], s_v)
    pltpu.sync_copy(x1_hbm.at[pl.ds(base, _PER_W)], x1_v)
    pltpu.sync_copy(y1_hbm.at[pl.ds(base, _PER_W)], y1_v)
    pltpu.sync_copy(x2_hbm.at[pl.ds(base, _PER_W)], x2_v)
    pltpu.sync_copy(y2_hbm.at[pl.ds(base, _PER_W)], y2_v)
    pltpu.sync_copy(lo_hbm, th_v.at[pl.ds(0, 16)])
    pltpu.sync_copy(him_hbm, th_v.at[pl.ds(16, 16)])
    lo_vec = th_v[pl.ds(0, 16)]
    him_vec = th_v[pl.ds(16, 16)].astype(jnp.int32)

    def init_body(i, _):
        sl = pl.ds(i * 16, 16)
        kb_v[sl] = jnp.full((16,), _PAD, jnp.float32)
        bx1_v[sl] = jnp.zeros((16,), jnp.float32)
        by1_v[sl] = jnp.zeros((16,), jnp.float32)
        bx2_v[sl] = jnp.zeros((16,), jnp.float32)
        by2_v[sl] = jnp.zeros((16,), jnp.float32)
        return 0

    lax.fori_loop(0, (_CAP + 16) // 16, init_body, 0)
    lane = lax.broadcasted_iota(jnp.int32, (16,), 0)

    def chunk_body(c, OFF):
        sl = pl.ds(c * 16, 16)
        s = s_v[sl]
        key = jnp.where(s > jnp.float32(0.05), s, jnp.float32(_NEG))
        idx = lane + (base + c * 16)
        lo_v = th_v[pl.ds(0, 16)]
        him_v = th_v[pl.ds(16, 16)].astype(jnp.int32)
        member = (key > lo_v) | ((key == lo_v) & (idx < him_v))
        inc = jnp.where(member, jnp.int32(1), jnp.int32(0))
        for kk in (1, 2, 4, 8):
            shifted = inc[jnp.maximum(lane - kk, 0)]
            inc = inc + jnp.where(lane >= kk, shifted, jnp.int32(0))
        pos = jnp.where(member, OFF + (inc - 1), jnp.int32(_CAP))
        pos = jnp.minimum(pos, jnp.int32(_CAP))
        plsc.store_scatter(kb_v, [pos], key)
        plsc.store_scatter(bx1_v, [pos], x1_v[sl])
        plsc.store_scatter(by1_v, [pos], y1_v[sl])
        plsc.store_scatter(bx2_v, [pos], x2_v[sl])
        plsc.store_scatter(by2_v, [pos], y2_v[sl])
        cnt = plsc.all_reduce_population_count(member)
        return jnp.minimum(OFF + cnt, jnp.full((16,), _CAP, jnp.int32))

    lax.fori_loop(0, _PER_W // 16, chunk_body, jnp.zeros((16,), jnp.int32))
    out = wid * _CAP
    pltpu.sync_copy(kb_v.at[pl.ds(0, _CAP)], pk_hbm.at[pl.ds(out, _CAP)])
    pltpu.sync_copy(bx1_v.at[pl.ds(0, _CAP)], px1_hbm.at[pl.ds(out, _CAP)])
    pltpu.sync_copy(by1_v.at[pl.ds(0, _CAP)], py1_hbm.at[pl.ds(out, _CAP)])
    pltpu.sync_copy(bx2_v.at[pl.ds(0, _CAP)], px2_hbm.at[pl.ds(out, _CAP)])
    pltpu.sync_copy(by2_v.at[pl.ds(0, _CAP)], py2_hbm.at[pl.ds(out, _CAP)])


def _greedy_body(k_ref, x1_ref, y1_ref, x2_ref, y2_ref, out_ref):
    kb = lax.bitcast_convert_type(k_ref[...], jnp.int32)
    x1 = x1_ref[...]
    y1 = y1_ref[...]
    x2 = x2_ref[...]
    y2 = y2_ref[...]

    ri = lax.broadcasted_iota(jnp.int32, (_PROWS, 128), 0)
    li = lax.broadcasted_iota(jnp.int32, (_PROWS, 128), 1)
    pos2d = ri * 128 + li
    pos = pos2d

    for kphase in range(1, 13):
        j = 1 << (kphase - 1)
        while j >= 1:
            if j < 128:
                axis, amt, n = 1, j, 128
            else:
                axis, amt, n = 0, j // 128, _PROWS
            lb = (pos2d & j) == 0
            up = (pos2d & (1 << kphase)) == 0

            def par(a, lb=lb, axis=axis, amt=amt, n=n):
                return jnp.where(lb, pltpu.roll(a, n - amt, axis),
                                 pltpu.roll(a, amt, axis))

            kbp = par(kb)
            posp = par(pos)
            a_first = (kb > kbp) | ((kb == kbp) & (pos < posp))
            ts = a_first == (lb == up)
            kb = jnp.where(ts, kb, kbp)
            pos = jnp.where(ts, pos, posp)
            x1 = jnp.where(ts, x1, par(x1))
            y1 = jnp.where(ts, y1, par(y1))
            x2 = jnp.where(ts, x2, par(x2))
            y2 = jnp.where(ts, y2, par(y2))
            j //= 2

    k0 = lax.bitcast_convert_type(kb, jnp.float32)
    area = (x2 - x1) * (y2 - y1)
    rowi = lax.broadcasted_iota(jnp.int32, (8, 128), 0)
    lanei = lax.broadcasted_iota(jnp.int32, (8, 128), 1)
    big = jnp.int32(2**30)

    def g_body(t, carry):
        k, acc = carry
        pa = jnp.where(k > jnp.float32(-5e8), pos2d, big)
        ps = jnp.where(k > jnp.float32(-1.5e9), pos2d, big)
        pa11 = jnp.min(jnp.min(pa, axis=1, keepdims=True), axis=0, keepdims=True)
        ps11 = jnp.min(jnp.min(ps, axis=1, keepdims=True), axis=0, keepdims=True)
        p11 = jnp.where(pa11 < big, pa11, ps11)
        P = jnp.broadcast_to(p11, (_PROWS, 128))
        onehot = pos2d == P
        z = jnp.float32(0.0)

        def pick(v):
            r = jnp.sum(jnp.where(onehot, v, z), axis=1, keepdims=True)
            return jnp.sum(r, axis=0, keepdims=True)

        bx1 = pick(x1)
        by1 = pick(y1)
        bx2 = pick(x2)
        by2 = pick(y2)
        ksel = pick(k)
        BX1 = jnp.broadcast_to(bx1, (_PROWS, 128))
        BY1 = jnp.broadcast_to(by1, (_PROWS, 128))
        BX2 = jnp.broadcast_to(bx2, (_PROWS, 128))
        BY2 = jnp.broadcast_to(by2, (_PROWS, 128))
        BAREA = (BX2 - BX1) * (BY2 - BY1)
        w = jnp.maximum(jnp.minimum(BX2, x2) - jnp.maximum(BX1, x1), z)
        h = jnp.maximum(jnp.minimum(BY2, y2) - jnp.maximum(BY1, y1), z)
        inter = w * h
        iou = inter / (((BAREA + area) - inter) + jnp.float32(1e-9))
        supp = (iou > jnp.float32(0.5)) & (k > jnp.float32(-5e8))
        nk = jnp.where(supp, k - jnp.float32(1e9), k)
        nk = jnp.where(onehot, jnp.float32(_DEAD), nk)
        outs = jnp.where(ksel > jnp.float32(-5e8), ksel, jnp.float32(_NEG))
        val = jnp.where(rowi == 0, jnp.broadcast_to(bx1, (8, 128)),
              jnp.where(rowi == 1, jnp.broadcast_to(by1, (8, 128)),
              jnp.where(rowi == 2, jnp.broadcast_to(bx2, (8, 128)),
              jnp.where(rowi == 3, jnp.broadcast_to(by2, (8, 128)),
                        jnp.broadcast_to(outs, (8, 128))))))
        acc = jnp.where(lanei == t, val, acc)
        return nk, acc

    acc0 = jnp.zeros((8, 128), jnp.float32)
    _, acc = lax.fori_loop(0, _DET, g_body, (k0, acc0))
    out_ref[...] = acc


_compact_call_cache = []


def _get_compact_call():
    if not _compact_call_cache:
        mesh = plsc.VectorSubcoreMesh(core_axis_name="c", subcore_axis_name="s",
                                      num_cores=2, num_subcores=16)
        _compact_call_cache.append(pl.kernel(
            _compact_body,
            out_type=(
                jax.ShapeDtypeStruct((_POOL,), jnp.float32),
                jax.ShapeDtypeStruct((_POOL,), jnp.float32),
                jax.ShapeDtypeStruct((_POOL,), jnp.float32),
                jax.ShapeDtypeStruct((_POOL,), jnp.float32),
                jax.ShapeDtypeStruct((_POOL,), jnp.float32),
            ),
            mesh=mesh,
            compiler_params=pltpu.CompilerParams(needs_layout_passes=False),
            scratch_types=[
                pltpu.VMEM((_PER_W,), jnp.float32),
                pltpu.VMEM((_PER_W,), jnp.float32),
                pltpu.VMEM((_PER_W,), jnp.float32),
                pltpu.VMEM((_PER_W,), jnp.float32),
                pltpu.VMEM((_PER_W,), jnp.float32),
                pltpu.VMEM((32,), jnp.float32),
                pltpu.VMEM((_CAP + 16,), jnp.float32),
                pltpu.VMEM((_CAP + 16,), jnp.float32),
                pltpu.VMEM((_CAP + 16,), jnp.float32),
                pltpu.VMEM((_CAP + 16,), jnp.float32),
                pltpu.VMEM((_CAP + 16,), jnp.float32),
            ],
        ))
    return _compact_call_cache[0]


def kernel(boxes, scores):
    s = jnp.pad(scores, (0, _NP - _N), constant_values=-1.0)
    b = jnp.pad(boxes, ((0, _NP - _N), (0, 0)))
    x1 = b[:, 0]
    y1 = b[:, 1]
    x2 = b[:, 2]
    y2 = b[:, 3]

    th = pl.pallas_call(
        _search_body,
        out_shape=jax.ShapeDtypeStruct((8, 128), jnp.float32),
    )(s.reshape(_ROWS, 128))
    lo16 = th[0, :16]
    him16 = th[1, :16]

    pk, px1, py1, px2, py2 = _get_compact_call()(s, x1, y1, x2, y2, lo16, him16)

    out = pl.pallas_call(
        _greedy_body,
        out_shape=jax.ShapeDtypeStruct((8, 128), jnp.float32),
    )(
        pk.reshape(_PROWS, 128),
        px1.reshape(_PROWS, 128),
        py1.reshape(_PROWS, 128),
        px2.reshape(_PROWS, 128),
        py2.reshape(_PROWS, 128),
    )
    return jnp.transpose(out[0:5, 0:_DET])


# speedup vs baseline: 1.0608x; 1.0608x over previous
import functools

import jax
import jax.numpy as jnp
from jax import lax
from jax.experimental import pallas as pl
from jax.experimental.pallas import tpu as pltpu
from jax.experimental.pallas import tpu_sc as plsc

_N = 20000
_ROWS = 160
_NP = _ROWS * 128
_K = 2000
_DET = 100
_NEG = -1e9
_PAD = -3e9
_DEAD = -2e9

_NW = 32
_PER_W = _NP // _NW
_CAP = 128
_POOL = _NW * _CAP
_PROWS = _POOL // 128


def _search_body(s_ref, out_ref):
    s = s_ref[...]
    key = jnp.where(s > jnp.float32(0.05), s, jnp.float32(_NEG))
    kbv = lax.bitcast_convert_type(key, jnp.int32)
    ridx = lax.broadcasted_iota(jnp.int32, (_ROWS, 128), 0)
    lidx = lax.broadcasted_iota(jnp.int32, (_ROWS, 128), 1)
    idx = ridx * 128 + lidx

    lo0 = lax.bitcast_convert_type(jnp.float32(0.04), jnp.int32)
    hi0 = lax.bitcast_convert_type(jnp.float32(1.5), jnp.int32)

    def bs_body(_, lh):
        lo, hi = lh
        mid = (lo + hi) // 2
        cnt = jnp.sum(jnp.where(kbv >= mid, 1, 0))
        pred = cnt >= _K
        return (jnp.where(pred, mid, lo), jnp.where(pred, hi, mid))

    lo, hi = lax.fori_loop(0, 26, bs_body, (lo0, hi0))

    n_gt = jnp.sum(jnp.where(kbv > lo, 1, 0))
    ties_needed = _K - n_gt

    def ib_body(_, lh):
        lom, him = lh
        mid = (lom + him) // 2
        cnt = jnp.sum(jnp.where((kbv == lo) & (idx < mid), 1, 0))
        pred = cnt >= ties_needed
        return (jnp.where(pred, lom, mid), jnp.where(pred, mid, him))

    _, him = lax.fori_loop(0, 15, ib_body, (jnp.int32(0), jnp.int32(_NP)))

    rowi = lax.broadcasted_iota(jnp.int32, (8, 128), 0)
    lo_f = lax.bitcast_convert_type(lo, jnp.float32)
    out_ref[...] = jnp.where(rowi == 0, lo_f, him.astype(jnp.float32))


def _compact_body(s_hbm, x1_hbm, y1_hbm, x2_hbm, y2_hbm, lo_hbm, him_hbm,
                  pk_hbm, px1_hbm, py1_hbm, px2_hbm, py2_hbm,
                  s_v, x1_v, y1_v, x2_v, y2_v, th_v,
                  kb_v, bx1_v, by1_v, bx2_v, by2_v):
    wid = lax.axis_index("s") * 2 + lax.axis_index("c")
    base = wid * _PER_W
    pltpu.sync_copy(s_hbm.at[pl.ds(base, _PER_W)], s_v)
    pltpu.sync_copy(x1_hbm.at[pl.ds(base, _PER_W)], x1_v)
    pltpu.sync_copy(y1_hbm.at[pl.ds(base, _PER_W)], y1_v)
    pltpu.sync_copy(x2_hbm.at[pl.ds(base, _PER_W)], x2_v)
    pltpu.sync_copy(y2_hbm.at[pl.ds(base, _PER_W)], y2_v)
    pltpu.sync_copy(lo_hbm, th_v.at[pl.ds(0, 16)])
    pltpu.sync_copy(him_hbm, th_v.at[pl.ds(16, 16)])
    lo_vec = th_v[pl.ds(0, 16)]
    him_vec = th_v[pl.ds(16, 16)].astype(jnp.int32)

    def init_body(i, _):
        sl = pl.ds(i * 16, 16)
        kb_v[sl] = jnp.full((16,), _PAD, jnp.float32)
        bx1_v[sl] = jnp.zeros((16,), jnp.float32)
        by1_v[sl] = jnp.zeros((16,), jnp.float32)
        bx2_v[sl] = jnp.zeros((16,), jnp.float32)
        by2_v[sl] = jnp.zeros((16,), jnp.float32)
        return 0

    lax.fori_loop(0, (_CAP + 16) // 16, init_body, 0)
    lane = lax.broadcasted_iota(jnp.int32, (16,), 0)

    def chunk_body(c, OFF):
        sl = pl.ds(c * 16, 16)
        s = s_v[sl]
        key = jnp.where(s > jnp.float32(0.05), s, jnp.float32(_NEG))
        idx = lane + (base + c * 16)
        lo_v = th_v[pl.ds(0, 16)]
        him_v = th_v[pl.ds(16, 16)].astype(jnp.int32)
        member = (key > lo_v) | ((key == lo_v) & (idx < him_v))
        inc = jnp.where(member, jnp.int32(1), jnp.int32(0))
        for kk in (1, 2, 4, 8):
            shifted = inc[jnp.maximum(lane - kk, 0)]
            inc = inc + jnp.where(lane >= kk, shifted, jnp.int32(0))
        pos = jnp.where(member, OFF + (inc - 1), jnp.int32(_CAP))
        pos = jnp.minimum(pos, jnp.int32(_CAP))
        plsc.store_scatter(kb_v, [pos], key)
        plsc.store_scatter(bx1_v, [pos], x1_v[sl])
        plsc.store_scatter(by1_v, [pos], y1_v[sl])
        plsc.store_scatter(bx2_v, [pos], x2_v[sl])
        plsc.store_scatter(by2_v, [pos], y2_v[sl])
        cnt = plsc.all_reduce_population_count(member)
        return jnp.minimum(OFF + cnt, jnp.full((16,), _CAP, jnp.int32))

    lax.fori_loop(0, _PER_W // 16, chunk_body, jnp.zeros((16,), jnp.int32))
    out = wid * _CAP
    pltpu.sync_copy(kb_v.at[pl.ds(0, _CAP)], pk_hbm.at[pl.ds(out, _CAP)])
    pltpu.sync_copy(bx1_v.at[pl.ds(0, _CAP)], px1_hbm.at[pl.ds(out, _CAP)])
    pltpu.sync_copy(by1_v.at[pl.ds(0, _CAP)], py1_hbm.at[pl.ds(out, _CAP)])
    pltpu.sync_copy(bx2_v.at[pl.ds(0, _CAP)], px2_hbm.at[pl.ds(out, _CAP)])
    pltpu.sync_copy(by2_v.at[pl.ds(0, _CAP)], py2_hbm.at[pl.ds(out, _CAP)])


def _sort_body(k_ref, x1_ref, y1_ref, x2_ref, y2_ref,
               ok_ref, ox1_ref, oy1_ref, ox2_ref, oy2_ref):
    kb = lax.bitcast_convert_type(k_ref[...], jnp.int32)
    x1 = x1_ref[...]
    y1 = y1_ref[...]
    x2 = x2_ref[...]
    y2 = y2_ref[...]

    ri = lax.broadcasted_iota(jnp.int32, (_PROWS, 128), 0)
    li = lax.broadcasted_iota(jnp.int32, (_PROWS, 128), 1)
    pos2d = ri * 128 + li
    pos = pos2d

    for kphase in range(1, 13):
        j = 1 << (kphase - 1)
        while j >= 1:
            if j < 128:
                axis, amt, n = 1, j, 128
            else:
                axis, amt, n = 0, j // 128, _PROWS
            lb = (pos2d & j) == 0
            up = (pos2d & (1 << kphase)) == 0

            def par(a, lb=lb, axis=axis, amt=amt, n=n):
                return jnp.where(lb, pltpu.roll(a, n - amt, axis),
                                 pltpu.roll(a, amt, axis))

            kbp = par(kb)
            posp = par(pos)
            a_first = (kb > kbp) | ((kb == kbp) & (pos < posp))
            ts = a_first == (lb == up)
            kb = jnp.where(ts, kb, kbp)
            pos = jnp.where(ts, pos, posp)
            x1 = jnp.where(ts, x1, par(x1))
            y1 = jnp.where(ts, y1, par(y1))
            x2 = jnp.where(ts, x2, par(x2))
            y2 = jnp.where(ts, y2, par(y2))
            j //= 2

    ok_ref[...] = lax.bitcast_convert_type(kb, jnp.float32)
    ox1_ref[...] = x1
    oy1_ref[...] = y1
    ox2_ref[...] = x2
    oy2_ref[...] = y2


def _pick_body(k_ref, x1_ref, y1_ref, x2_ref, y2_ref,
               ks_ref, x1s_ref, y1s_ref, x2s_ref, y2s_ref, out_ref):
    x1 = x1_ref[...]
    y1 = y1_ref[...]
    x2 = x2_ref[...]
    y2 = y2_ref[...]
    area = (x2 - x1) * (y2 - y1)

    ri = lax.broadcasted_iota(jnp.int32, (_PROWS, 128), 0)
    li = lax.broadcasted_iota(jnp.int32, (_PROWS, 128), 1)
    pos2d = ri * 128 + li
    rowi = lax.broadcasted_iota(jnp.int32, (8, 128), 0)
    lanei = lax.broadcasted_iota(jnp.int32, (8, 128), 1)
    big = jnp.int32(2**30)

    def g_body(t, carry):
        k, acc = carry
        pa = jnp.where(k > jnp.float32(-5e8), pos2d, big)
        ps = jnp.where(k > jnp.float32(-1.5e9), pos2d, big)
        pa11 = jnp.min(jnp.min(pa, axis=0, keepdims=True), axis=1, keepdims=True)
        ps11 = jnp.min(jnp.min(ps, axis=0, keepdims=True), axis=1, keepdims=True)
        spa = pa11[0, 0]
        sps = ps11[0, 0]
        sp = jnp.where(spa < big, spa, sps)
        bx1 = x1s_ref[sp]
        by1 = y1s_ref[sp]
        bx2 = x2s_ref[sp]
        by2 = y2s_ref[sp]
        bsc = ks_ref[sp]
        barea = (bx2 - bx1) * (by2 - by1)
        z = jnp.float32(0.0)
        w = jnp.maximum(jnp.minimum(bx2, x2) - jnp.maximum(bx1, x1), z)
        h = jnp.maximum(jnp.minimum(by2, y2) - jnp.maximum(by1, y1), z)
        inter = w * h
        iou = inter / (((barea + area) - inter) + jnp.float32(1e-9))
        supp = (iou > jnp.float32(0.5)) & (k > jnp.float32(-5e8))
        nk = jnp.where(supp, k - jnp.float32(1e9), k)
        nk = jnp.where(pos2d == sp, jnp.float32(_DEAD), nk)
        outs = jnp.where(bsc > jnp.float32(-5e8), bsc, jnp.float32(_NEG))
        val = jnp.where(rowi == 0, bx1,
              jnp.where(rowi == 1, by1,
              jnp.where(rowi == 2, bx2,
              jnp.where(rowi == 3, by2, outs))))
        acc = jnp.where(lanei == t, val, acc)
        return nk, acc

    acc0 = jnp.zeros((8, 128), jnp.float32)
    _, acc = lax.fori_loop(0, _DET, g_body, (k_ref[...], acc0))
    out_ref[...] = acc


_compact_call_cache = []


def _get_compact_call():
    if not _compact_call_cache:
        mesh = plsc.VectorSubcoreMesh(core_axis_name="c", subcore_axis_name="s",
                                      num_cores=2, num_subcores=16)
        _compact_call_cache.append(pl.kernel(
            _compact_body,
            out_type=(
                jax.ShapeDtypeStruct((_POOL,), jnp.float32),
                jax.ShapeDtypeStruct((_POOL,), jnp.float32),
                jax.ShapeDtypeStruct((_POOL,), jnp.float32),
                jax.ShapeDtypeStruct((_POOL,), jnp.float32),
                jax.ShapeDtypeStruct((_POOL,), jnp.float32),
            ),
            mesh=mesh,
            compiler_params=pltpu.CompilerParams(needs_layout_passes=False),
            scratch_types=[
                pltpu.VMEM((_PER_W,), jnp.float32),
                pltpu.VMEM((_PER_W,), jnp.float32),
                pltpu.VMEM((_PER_W,), jnp.float32),
                pltpu.VMEM((_PER_W,), jnp.float32),
                pltpu.VMEM((_PER_W,), jnp.float32),
                pltpu.VMEM((32,), jnp.float32),
                pltpu.VMEM((_CAP + 16,), jnp.float32),
                pltpu.VMEM((_CAP + 16,), jnp.float32),
                pltpu.VMEM((_CAP + 16,), jnp.float32),
                pltpu.VMEM((_CAP + 16,), jnp.float32),
                pltpu.VMEM((_CAP + 16,), jnp.float32),
            ],
        ))
    return _compact_call_cache[0]


def kernel(boxes, scores):
    s = jnp.pad(scores, (0, _NP - _N), constant_values=-1.0)
    b = jnp.pad(boxes, ((0, _NP - _N), (0, 0)))
    x1 = b[:, 0]
    y1 = b[:, 1]
    x2 = b[:, 2]
    y2 = b[:, 3]

    th = pl.pallas_call(
        _search_body,
        out_shape=jax.ShapeDtypeStruct((8, 128), jnp.float32),
    )(s.reshape(_ROWS, 128))
    lo16 = th[0, :16]
    him16 = th[1, :16]

    pk, px1, py1, px2, py2 = _get_compact_call()(s, x1, y1, x2, y2, lo16, him16)

    shp = jax.ShapeDtypeStruct((_PROWS, 128), jnp.float32)
    sk, sx1, sy1, sx2, sy2 = pl.pallas_call(
        _sort_body,
        out_shape=[shp] * 5,
    )(
        pk.reshape(_PROWS, 128),
        px1.reshape(_PROWS, 128),
        py1.reshape(_PROWS, 128),
        px2.reshape(_PROWS, 128),
        py2.reshape(_PROWS, 128),
    )

    vspec = pl.BlockSpec(memory_space=pltpu.VMEM)
    sspec = pl.BlockSpec(memory_space=pltpu.SMEM)
    out = pl.pallas_call(
        _pick_body,
        out_shape=jax.ShapeDtypeStruct((8, 128), jnp.float32),
        in_specs=[vspec] * 5 + [sspec] * 5,
    )(
        sk, sx1, sy1, sx2, sy2,
        sk.reshape(_POOL), sx1.reshape(_POOL), sy1.reshape(_POOL),
        sx2.reshape(_POOL), sy2.reshape(_POOL),
    )
    return jnp.transpose(out[0:5, 0:_DET])
